# baseline (speedup 1.0000x reference)
import jax
import jax.numpy as jnp
from jax.experimental import pallas as pl
from jax.experimental.pallas import tpu as pltpu

_Q = 1024
_D = 768
_K = 100000
_BK = 5120
_NB = 20
_HALF = _BK // 4
_CPH = _HALF // 128
_NEG = -3.4e38


def _topk_kernel(q_ref, k_ref, val_ref, idx_ref, R_ref):
    j = pl.program_id(0)

    @pl.when(j == 0)
    def _init():
        R_ref[...] = jnp.full((_Q, 128), _NEG, jnp.float32)

    R = R_ref[...]
    for h in range(4):
        kh = k_ref[h * _HALF:(h + 1) * _HALF, :]
        s = jax.lax.dot_general(
            q_ref[...], kh,
            dimension_numbers=(((1,), (1,)), ((), ())),
            preferred_element_type=jnp.float32,
        )
        for c in range(_CPH):
            sc = jax.lax.slice_in_dim(s, c * 128, (c + 1) * 128, axis=1)
            R = jnp.maximum(R, sc)
    R_ref[...] = R

    @pl.when(j == _NB - 1)
    def _done():
        v = jnp.max(R_ref[...], axis=1, keepdims=True)
        val_ref[...] = v
        idx_ref[...] = jnp.zeros((_Q, 1), jnp.int32)


def kernel(queries, keys):
    top_vals, top_idx = pl.pallas_call(
        _topk_kernel,
        grid=(_NB,),
        in_specs=[
            pl.BlockSpec((_Q, _D), lambda j: (0, 0)),
            pl.BlockSpec((_BK, _D), lambda j: (j, 0)),
        ],
        out_specs=[
            pl.BlockSpec((_Q, 1), lambda j: (0, 0)),
            pl.BlockSpec((_Q, 1), lambda j: (0, 0)),
        ],
        out_shape=[
            jax.ShapeDtypeStruct((_Q, 1), jnp.float32),
            jax.ShapeDtypeStruct((_Q, 1), jnp.int32),
        ],
        scratch_shapes=[
            pltpu.VMEM((_Q, 128), jnp.float32),
        ],
        compiler_params=pltpu.CompilerParams(
            dimension_semantics=("arbitrary",),
        ),
    )(queries, keys)
    return top_vals, top_idx


# speedup vs baseline: 1.0175x; 1.0175x over previous
import jax
import jax.numpy as jnp
from jax.experimental import pallas as pl
from jax.experimental.pallas import tpu as pltpu

_Q = 1024
_D = 768
_K = 100000
_BK = 2560
_CPB = _BK // 128
_NB = 20
_NEG = -3.4e38
_IMAX = 2147483647


def _fold(s, chunk0, nchunks, R, T, first_masked_lanes=None):
    lane = jax.lax.broadcasted_iota(jnp.int32, (_Q, 128), 1)
    for c in range(nchunks):
        sc = jax.lax.slice_in_dim(s, c * 128, (c + 1) * 128, axis=1)
        if first_masked_lanes is not None and c == nchunks - 1:
            sc = jnp.where(lane < first_masked_lanes, sc, _NEG)
        upd = sc > R
        R = jnp.where(upd, sc, R)
        T = jnp.where(upd, jnp.int32(chunk0 + c), T)
    return R, T


def _dot(q, k):
    return jax.lax.dot_general(
        q, k,
        dimension_numbers=(((1,), (1,)), ((), ())),
        preferred_element_type=jnp.float32,
    )


def _topk_kernel(q_ref, ka_ref, kb_ref, val_ref, idx_ref, R_ref, T_ref):
    j = pl.program_id(0)

    @pl.when(j == 0)
    def _init():
        R_ref[...] = jnp.full((_Q, 128), _NEG, jnp.float32)
        T_ref[...] = jnp.zeros((_Q, 128), jnp.int32)

    @pl.when(j < _NB - 1)
    def _full_block():
        R = R_ref[...]
        T = T_ref[...]
        s = _dot(q_ref[...], ka_ref[...])
        R, T = _fold(s, (2 * j) * _CPB, _CPB, R, T)
        s = _dot(q_ref[...], kb_ref[...])
        R, T = _fold(s, (2 * j + 1) * _CPB, _CPB, R, T)
        R_ref[...] = R
        T_ref[...] = T

    @pl.when(j == _NB - 1)
    def _tail_block():
        R = R_ref[...]
        T = T_ref[...]
        s = _dot(q_ref[...], ka_ref[...])
        R, T = _fold(s, (2 * j) * _CPB, _CPB, R, T)
        s = _dot(q_ref[...], kb_ref[...])
        valid = _K - (2 * _NB - 1) * _BK
        vchunks = valid // 128
        rag = valid - vchunks * 128
        chunk0 = (2 * j + 1) * _CPB
        R, T = _fold(s, chunk0, vchunks, R, T)
        R, T = _fold(
            jax.lax.slice_in_dim(s, vchunks * 128, (vchunks + 1) * 128,
                                 axis=1),
            chunk0 + vchunks, 1, R, T, first_masked_lanes=rag)

        v = jnp.max(R, axis=1, keepdims=True)
        lane = jax.lax.broadcasted_iota(jnp.int32, (_Q, 128), 1)
        gidx = T * 128 + lane
        idxv = jnp.min(jnp.where(R == v, gidx, _IMAX), axis=1, keepdims=True)
        val_ref[...] = v
        idx_ref[...] = idxv


def kernel(queries, keys):
    top_vals, top_idx = pl.pallas_call(
        _topk_kernel,
        grid=(_NB,),
        in_specs=[
            pl.BlockSpec((_Q, _D), lambda j: (0, 0)),
            pl.BlockSpec((_BK, _D), lambda j: (2 * j, 0)),
            pl.BlockSpec((_BK, _D), lambda j: (2 * j + 1, 0)),
        ],
        out_specs=[
            pl.BlockSpec((_Q, 1), lambda j: (0, 0)),
            pl.BlockSpec((_Q, 1), lambda j: (0, 0)),
        ],
        out_shape=[
            jax.ShapeDtypeStruct((_Q, 1), jnp.float32),
            jax.ShapeDtypeStruct((_Q, 1), jnp.int32),
        ],
        scratch_shapes=[
            pltpu.VMEM((_Q, 128), jnp.float32),
            pltpu.VMEM((_Q, 128), jnp.int32),
        ],
        compiler_params=pltpu.CompilerParams(
            dimension_semantics=("arbitrary",),
        ),
    )(queries, keys, keys)
    return top_vals, top_idx
